# Initial kernel scaffold; baseline (speedup 1.0000x reference)
import jax
import jax.numpy as jnp
from jax.experimental import pallas as pl


def kernel(x, input_ids, attention_mask, ln_g, ln_b, W1, b1, W2, W3, noise):
    raise NotImplementedError("write your pallas kernel here")


# speedup vs baseline: 24.0790x; 24.0790x over previous
import functools

import jax
import jax.numpy as jnp
from jax.experimental import pallas as pl

EMBED_DIM = 768
TOPK = 8
NUM_SAMPLES = 500
SIGMA = 0.05
B, N = 4, 2048
C = EMBED_DIM // 2

_INT_MIN = jnp.iinfo(jnp.int32).min
_INV_SQRT2 = 0.7071067811865476


def _gelu_exact(v):
    return v * 0.5 * (1.0 + jax.lax.erf(v * _INV_SQRT2))

_SORT8 = [(0, 1), (2, 3), (4, 5), (6, 7),
          (0, 2), (1, 3), (4, 6), (5, 7),
          (1, 2), (5, 6),
          (0, 4), (1, 5), (2, 6), (3, 7),
          (2, 4), (3, 5),
          (1, 2), (3, 4), (5, 6)]


def _scores_kernel(x_ref, ids_ref, amn_ref, lng_ref, lnb_ref, w1_ref, b1_ref,
                   w2a_ref, w2b_ref, w3_ref, score_ref, cls_ref):
    xb = x_ref[0]
    mu = jnp.mean(xb, axis=-1, keepdims=True)
    xc = xb - mu
    var = jnp.mean(xc * xc, axis=-1, keepdims=True)
    ln = xc / jnp.sqrt(var + 1e-5) * lng_ref[0][None, :] + lnb_ref[0][None, :]
    h = _gelu_exact(
        jnp.dot(ln, w1_ref[...], preferred_element_type=jnp.float32)
        + b1_ref[0][None, :])

    ids = ids_ref[0]
    iota = jax.lax.broadcasted_iota(jnp.int32, (1, N), 1)
    ikey = ids * N + (N - 1 - iota)
    gmax = jnp.max(ikey)
    gsel = (ikey == gmax).astype(jnp.float32)
    h_g = jnp.dot(gsel, h, preferred_element_type=jnp.float32)
    cls_ref[0] = jnp.dot(gsel, xb, preferred_element_type=jnp.float32)

    bias2 = jnp.dot(h_g, w2b_ref[...], preferred_element_type=jnp.float32)
    o = _gelu_exact(
        jnp.dot(h, w2a_ref[...], preferred_element_type=jnp.float32) + bias2)
    s = jax.nn.sigmoid(
        jnp.dot(o, w3_ref[...], preferred_element_type=jnp.float32))
    score_ref[0] = s.reshape(1, N) * amn_ref[0]


def _topk_counts_kernel(noise_ref, score_ref, counts_ref):
    sc = score_ref[0]
    p = sc + noise_ref[0] * SIGMA
    q = ((p + 0.5) * 262144.0).astype(jnp.int32)
    iota = jax.lax.broadcasted_iota(jnp.int32, (1, N), 1)
    keys = q * N + (N - 1 - iota)

    idxs = []
    for _ in range(TOPK):
        m = jnp.max(keys, axis=1, keepdims=True)
        idxs.append((N - 1) - (m & (N - 1)))
        keys = jnp.where(keys == m, _INT_MIN, keys)

    for a, b in _SORT8:
        lo = jnp.minimum(idxs[a], idxs[b])
        hi = jnp.maximum(idxs[a], idxs[b])
        idxs[a], idxs[b] = lo, hi

    for k in range(TOPK):
        eq = (idxs[k] == iota).astype(jnp.float32)
        counts_ref[0, k] = jnp.sum(eq, axis=0)


def _select_kernel(counts_ref, x_ref, cls_ref, out_ref):
    sel = jnp.dot(counts_ref[0], x_ref[0],
                  preferred_element_type=jnp.float32) * (1.0 / NUM_SAMPLES)
    out_ref[0, 0] = cls_ref[0, 0]
    out_ref[0, 1:] = sel


@jax.jit
def kernel(x, input_ids, attention_mask, ln_g, ln_b, W1, b1, W2, W3, noise):
    Bn, Nn, D = x.shape
    am_new = jnp.concatenate(
        [attention_mask[:, 1:], jnp.zeros((Bn, 1), attention_mask.dtype)],
        axis=1)
    ids3 = input_ids.reshape(Bn, 1, Nn)
    amn3 = am_new.reshape(Bn, 1, Nn)
    lng2 = ln_g.reshape(1, D)
    lnb2 = ln_b.reshape(1, D)
    b12 = b1.reshape(1, C)
    W2a = W2[:C]
    W2b = W2[C:]

    score, cls = pl.pallas_call(
        _scores_kernel,
        grid=(Bn,),
        in_specs=[
            pl.BlockSpec((1, Nn, D), lambda b: (b, 0, 0)),
            pl.BlockSpec((1, 1, Nn), lambda b: (b, 0, 0)),
            pl.BlockSpec((1, 1, Nn), lambda b: (b, 0, 0)),
            pl.BlockSpec((1, D), lambda b: (0, 0)),
            pl.BlockSpec((1, D), lambda b: (0, 0)),
            pl.BlockSpec((D, C), lambda b: (0, 0)),
            pl.BlockSpec((1, C), lambda b: (0, 0)),
            pl.BlockSpec((C, C), lambda b: (0, 0)),
            pl.BlockSpec((C, C), lambda b: (0, 0)),
            pl.BlockSpec((C, 1), lambda b: (0, 0)),
        ],
        out_specs=[
            pl.BlockSpec((1, 1, Nn), lambda b: (b, 0, 0)),
            pl.BlockSpec((1, 1, D), lambda b: (b, 0, 0)),
        ],
        out_shape=[
            jax.ShapeDtypeStruct((Bn, 1, Nn), jnp.float32),
            jax.ShapeDtypeStruct((Bn, 1, D), jnp.float32),
        ],
    )(x, ids3, amn3, lng2, lnb2, W1, b12, W2a, W2b, W3)

    counts = pl.pallas_call(
        _topk_counts_kernel,
        grid=(Bn,),
        in_specs=[
            pl.BlockSpec((1, NUM_SAMPLES, Nn), lambda b: (b, 0, 0)),
            pl.BlockSpec((1, 1, Nn), lambda b: (b, 0, 0)),
        ],
        out_specs=pl.BlockSpec((1, TOPK, Nn), lambda b: (b, 0, 0)),
        out_shape=jax.ShapeDtypeStruct((Bn, TOPK, Nn), jnp.float32),
    )(noise, score)

    out = pl.pallas_call(
        _select_kernel,
        grid=(Bn,),
        in_specs=[
            pl.BlockSpec((1, TOPK, Nn), lambda b: (b, 0, 0)),
            pl.BlockSpec((1, Nn, D), lambda b: (b, 0, 0)),
            pl.BlockSpec((1, 1, D), lambda b: (b, 0, 0)),
        ],
        out_specs=pl.BlockSpec((1, 1 + TOPK, D), lambda b: (b, 0, 0)),
        out_shape=jax.ShapeDtypeStruct((Bn, 1 + TOPK, D), jnp.float32),
    )(counts, x, cls)
    return out
